# baseline (speedup 1.0000x reference)
import jax
import jax.numpy as jnp
from jax.experimental import pallas as pl
from jax.experimental.pallas import tpu as pltpu

_B, _S, _D, _E = 4, 2048, 1024, 3
_S_BLK = 512
_S_BLKS = _S // _S_BLK
_ANY = pl.ANY


def _body(v0_ref, a0_ref, w_ref, bias_ref, v_ref, a_ref, av_ref,
          logits_ref, out_ref, acc_ref, sem):
    bi = pl.program_id(0)
    j = pl.program_id(1)

    @pl.when(j == 0)
    def _():
        acc_ref[...] = jnp.zeros_like(acc_ref)

    vb = v0_ref[0]
    ab = a0_ref[0]
    acc_ref[0, :] += jnp.sum(vb, axis=0)
    acc_ref[1, :] += jnp.sum(vb * vb, axis=0)
    acc_ref[2, :] += jnp.sum(ab, axis=0)
    acc_ref[3, :] += jnp.sum(ab * ab, axis=0)

    @pl.when(j == _S_BLKS - 1)
    def _():
        inv_s = 1.0 / _S
        inv_n1 = 1.0 / (_S - 1)
        mean_v = acc_ref[0:1, :] * inv_s
        var_v = (acc_ref[1:2, :] - _S * mean_v * mean_v) * inv_n1
        mean_a = acc_ref[2:3, :] * inv_s
        var_a = (acc_ref[3:4, :] - _S * mean_a * mean_a) * inv_n1
        feats = jnp.concatenate(
            [mean_v, jnp.sqrt(var_v), mean_a, jnp.sqrt(var_a)], axis=1
        )
        logits = jnp.sum(w_ref[...] * feats, axis=1) + bias_ref[0]
        logits_ref[0, 0, :] = logits

        l0, l1, l2 = logits[0], logits[1], logits[2]
        e = jnp.where(l2 > jnp.maximum(l0, l1), 2, jnp.where(l1 > l0, 1, 0))

        @pl.when(e == 0)
        def _():
            pltpu.make_async_copy(v_ref.at[bi], out_ref.at[bi], sem).start()

        @pl.when(e == 1)
        def _():
            pltpu.make_async_copy(a_ref.at[bi], out_ref.at[bi], sem).start()

        @pl.when(e == 2)
        def _():
            pltpu.make_async_copy(av_ref.at[bi], out_ref.at[bi], sem).start()

    @pl.when((bi == _B - 1) & (j == _S_BLKS - 1))
    def _():
        for k in range(_B):
            pltpu.make_async_copy(v_ref.at[k], out_ref.at[k], sem).wait()


def kernel(v0, a0, v, a, av, W, b):
    logits3, combined = pl.pallas_call(
        _body,
        grid=(_B, _S_BLKS),
        in_specs=[
            pl.BlockSpec((1, _S_BLK, _D), lambda bi, j: (bi, j, 0)),
            pl.BlockSpec((1, _S_BLK, _D), lambda bi, j: (bi, j, 0)),
            pl.BlockSpec((_E, 4 * _D), lambda bi, j: (0, 0)),
            pl.BlockSpec((1, _E), lambda bi, j: (0, 0)),
            pl.BlockSpec(memory_space=_ANY),
            pl.BlockSpec(memory_space=_ANY),
            pl.BlockSpec(memory_space=_ANY),
        ],
        out_specs=[
            pl.BlockSpec((1, 1, _E), lambda bi, j: (bi, 0, 0)),
            pl.BlockSpec(memory_space=_ANY),
        ],
        out_shape=[
            jax.ShapeDtypeStruct((_B, 1, _E), jnp.float32),
            jax.ShapeDtypeStruct((_B, _S, _D), jnp.float32),
        ],
        scratch_shapes=[
            pltpu.VMEM((8, _D), jnp.float32),
            pltpu.SemaphoreType.DMA,
        ],
        compiler_params=pltpu.CompilerParams(
            dimension_semantics=("arbitrary", "arbitrary")
        ),
    )(v0, a0, W, b.reshape(1, _E), v, a, av)

    return combined, logits3.reshape(_B, _E)


# speedup vs baseline: 21.1771x; 21.1771x over previous
import jax
import jax.numpy as jnp
from jax.experimental import pallas as pl
from jax.experimental.pallas import tpu as pltpu

_B, _S, _D, _E = 4, 2048, 1024, 3
_S_BLK = 1024
_S_BLKS = _S // _S_BLK

_CH = 256
_CHUNKS = _S // _CH
_TOTAL = _B * _CHUNKS
_NBUF = 8
_LAG = 4


def _stats_body(v0_ref, a0_ref, w_ref, bias_ref, logits_ref, acc_ref):
    j = pl.program_id(1)

    @pl.when(j == 0)
    def _():
        acc_ref[...] = jnp.zeros_like(acc_ref)

    vb = v0_ref[0]
    ab = a0_ref[0]
    acc_ref[0, :] += jnp.sum(vb, axis=0)
    acc_ref[1, :] += jnp.sum(vb * vb, axis=0)
    acc_ref[2, :] += jnp.sum(ab, axis=0)
    acc_ref[3, :] += jnp.sum(ab * ab, axis=0)

    @pl.when(j == _S_BLKS - 1)
    def _():
        inv_s = 1.0 / _S
        inv_n1 = 1.0 / (_S - 1)
        mean_v = acc_ref[0:1, :] * inv_s
        var_v = (acc_ref[1:2, :] - _S * mean_v * mean_v) * inv_n1
        mean_a = acc_ref[2:3, :] * inv_s
        var_a = (acc_ref[3:4, :] - _S * mean_a * mean_a) * inv_n1
        feats = jnp.concatenate(
            [mean_v, jnp.sqrt(var_v), mean_a, jnp.sqrt(var_a)], axis=1
        )
        logits = jnp.sum(w_ref[...] * feats, axis=1) + bias_ref[0]
        logits_ref[0, 0, :] = logits


def _argmax3(lg_ref, b):
    l0 = lg_ref[3 * b]
    l1 = lg_ref[3 * b + 1]
    l2 = lg_ref[3 * b + 2]
    i01 = jnp.where(l1 > l0, 1, 0)
    m01 = jnp.maximum(l0, l1)
    return jnp.where(l2 > m01, 2, i01)


def _copy_body(lg_ref, v_ref, a_ref, av_ref, out_ref, buf_ref, rsem, wsem):
    es = [_argmax3(lg_ref, b) for b in range(_B)]

    def chunk(t):
        b, j = divmod(t, _CHUNKS)
        return b, pl.ds(j * _CH, _CH), t % _NBUF

    def read_start(t):
        b, rows, slot = chunk(t)
        for e, src in ((0, v_ref), (1, a_ref), (2, av_ref)):
            @pl.when(es[b] == e)
            def _(src=src):
                pltpu.make_async_copy(
                    src.at[b, rows, :], buf_ref.at[slot], rsem.at[slot]
                ).start()

    def read_wait(t):
        b, rows, slot = chunk(t)
        pltpu.make_async_copy(
            v_ref.at[b, rows, :], buf_ref.at[slot], rsem.at[slot]
        ).wait()

    def write_copy(t):
        b, rows, slot = chunk(t)
        return pltpu.make_async_copy(
            buf_ref.at[slot], out_ref.at[b, rows, :], wsem.at[slot]
        )

    for t in range(_TOTAL + _LAG):
        if t < _TOTAL:
            if t >= _NBUF:
                write_copy(t - _NBUF).wait()
            read_start(t)
        if t >= _LAG:
            read_wait(t - _LAG)
            write_copy(t - _LAG).start()
    for t in range(_TOTAL - _NBUF, _TOTAL):
        write_copy(t).wait()


def kernel(v0, a0, v, a, av, W, b):
    logits3 = pl.pallas_call(
        _stats_body,
        grid=(_B, _S_BLKS),
        in_specs=[
            pl.BlockSpec((1, _S_BLK, _D), lambda bi, j: (bi, j, 0)),
            pl.BlockSpec((1, _S_BLK, _D), lambda bi, j: (bi, j, 0)),
            pl.BlockSpec((_E, 4 * _D), lambda bi, j: (0, 0)),
            pl.BlockSpec((1, _E), lambda bi, j: (0, 0)),
        ],
        out_specs=pl.BlockSpec((1, 1, _E), lambda bi, j: (bi, 0, 0)),
        out_shape=jax.ShapeDtypeStruct((_B, 1, _E), jnp.float32),
        scratch_shapes=[pltpu.VMEM((8, _D), jnp.float32)],
        compiler_params=pltpu.CompilerParams(
            dimension_semantics=("arbitrary", "arbitrary")
        ),
    )(v0, a0, W, b.reshape(1, _E))

    logits = logits3.reshape(_B, _E)

    combined = pl.pallas_call(
        _copy_body,
        grid_spec=pltpu.PrefetchScalarGridSpec(
            num_scalar_prefetch=1,
            grid=(1,),
            in_specs=[
                pl.BlockSpec(memory_space=pl.ANY),
                pl.BlockSpec(memory_space=pl.ANY),
                pl.BlockSpec(memory_space=pl.ANY),
            ],
            out_specs=pl.BlockSpec(memory_space=pl.ANY),
            scratch_shapes=[
                pltpu.VMEM((_NBUF, _CH, _D), jnp.float32),
                pltpu.SemaphoreType.DMA((_NBUF,)),
                pltpu.SemaphoreType.DMA((_NBUF,)),
            ],
        ),
        out_shape=jax.ShapeDtypeStruct((_B, _S, _D), jnp.float32),
    )(logits.reshape(_B * _E), v, a, av)

    return combined, logits


# speedup vs baseline: 21.3727x; 1.0092x over previous
import jax
import jax.numpy as jnp
from jax.experimental import pallas as pl
from jax.experimental.pallas import tpu as pltpu

_B, _S, _D, _E = 4, 2048, 1024, 3
_S_BLK = 1024
_S_BLKS = _S // _S_BLK

_CH = 256
_CHUNKS = _S // _CH
_TOTAL = _B * _CHUNKS
_NBUF = 8
_LAG = 4


_SCH = 256
_SCHUNKS = _S // _SCH
_STOTAL = _B * _SCHUNKS
_SNBUF = 6


def _stats_body(v0_ref, a0_ref, w_ref, bias_ref, logits_ref,
                vbuf_ref, abuf_ref, acc_ref, vsem, asem):
    def read_start(t):
        b, j = divmod(t, _SCHUNKS)
        rows = pl.ds(j * _SCH, _SCH)
        slot = t % _SNBUF
        pltpu.make_async_copy(
            v0_ref.at[b, rows, :], vbuf_ref.at[slot], vsem.at[slot]
        ).start()
        pltpu.make_async_copy(
            a0_ref.at[b, rows, :], abuf_ref.at[slot], asem.at[slot]
        ).start()

    def read_wait(t):
        b, j = divmod(t, _SCHUNKS)
        rows = pl.ds(j * _SCH, _SCH)
        slot = t % _SNBUF
        pltpu.make_async_copy(
            v0_ref.at[b, rows, :], vbuf_ref.at[slot], vsem.at[slot]
        ).wait()
        pltpu.make_async_copy(
            a0_ref.at[b, rows, :], abuf_ref.at[slot], asem.at[slot]
        ).wait()

    for t in range(min(_SNBUF, _STOTAL)):
        read_start(t)

    for t in range(_STOTAL):
        b, j = divmod(t, _SCHUNKS)
        slot = t % _SNBUF
        read_wait(t)
        vb = vbuf_ref[slot]
        ab = abuf_ref[slot]
        if j == 0:
            acc_ref[0, :] = jnp.sum(vb, axis=0)
            acc_ref[1, :] = jnp.sum(vb * vb, axis=0)
            acc_ref[2, :] = jnp.sum(ab, axis=0)
            acc_ref[3, :] = jnp.sum(ab * ab, axis=0)
        else:
            acc_ref[0, :] += jnp.sum(vb, axis=0)
            acc_ref[1, :] += jnp.sum(vb * vb, axis=0)
            acc_ref[2, :] += jnp.sum(ab, axis=0)
            acc_ref[3, :] += jnp.sum(ab * ab, axis=0)
        if t + _SNBUF < _STOTAL:
            read_start(t + _SNBUF)
        if j == _SCHUNKS - 1:
            inv_s = 1.0 / _S
            inv_n1 = 1.0 / (_S - 1)
            mean_v = acc_ref[0:1, :] * inv_s
            var_v = (acc_ref[1:2, :] - _S * mean_v * mean_v) * inv_n1
            mean_a = acc_ref[2:3, :] * inv_s
            var_a = (acc_ref[3:4, :] - _S * mean_a * mean_a) * inv_n1
            feats = jnp.concatenate(
                [mean_v, jnp.sqrt(var_v), mean_a, jnp.sqrt(var_a)], axis=1
            )
            logits = jnp.sum(w_ref[...] * feats, axis=1) + bias_ref[0]
            logits_ref[b, :] = logits


def _argmax3(lg_ref, b):
    l0 = lg_ref[3 * b]
    l1 = lg_ref[3 * b + 1]
    l2 = lg_ref[3 * b + 2]
    i01 = jnp.where(l1 > l0, 1, 0)
    m01 = jnp.maximum(l0, l1)
    return jnp.where(l2 > m01, 2, i01)


def _copy_body(lg_ref, v_ref, a_ref, av_ref, out_ref, buf_ref, rsem, wsem):
    es = [_argmax3(lg_ref, b) for b in range(_B)]

    def chunk(t):
        b, j = divmod(t, _CHUNKS)
        return b, pl.ds(j * _CH, _CH), t % _NBUF

    def read_start(t):
        b, rows, slot = chunk(t)
        for e, src in ((0, v_ref), (1, a_ref), (2, av_ref)):
            @pl.when(es[b] == e)
            def _(src=src):
                pltpu.make_async_copy(
                    src.at[b, rows, :], buf_ref.at[slot], rsem.at[slot]
                ).start()

    def read_wait(t):
        b, rows, slot = chunk(t)
        pltpu.make_async_copy(
            v_ref.at[b, rows, :], buf_ref.at[slot], rsem.at[slot]
        ).wait()

    def write_copy(t):
        b, rows, slot = chunk(t)
        return pltpu.make_async_copy(
            buf_ref.at[slot], out_ref.at[b, rows, :], wsem.at[slot]
        )

    for t in range(_TOTAL + _LAG):
        if t < _TOTAL:
            if t >= _NBUF:
                write_copy(t - _NBUF).wait()
            read_start(t)
        if t >= _LAG:
            read_wait(t - _LAG)
            write_copy(t - _LAG).start()
    for t in range(_TOTAL - _NBUF, _TOTAL):
        write_copy(t).wait()


def kernel(v0, a0, v, a, av, W, b):
    logits = pl.pallas_call(
        _stats_body,
        grid=(1,),
        in_specs=[
            pl.BlockSpec(memory_space=pl.ANY),
            pl.BlockSpec(memory_space=pl.ANY),
            pl.BlockSpec((_E, 4 * _D), lambda i: (0, 0)),
            pl.BlockSpec((1, _E), lambda i: (0, 0)),
        ],
        out_specs=pl.BlockSpec((_B, _E), lambda i: (0, 0)),
        out_shape=jax.ShapeDtypeStruct((_B, _E), jnp.float32),
        scratch_shapes=[
            pltpu.VMEM((_SNBUF, _SCH, _D), jnp.float32),
            pltpu.VMEM((_SNBUF, _SCH, _D), jnp.float32),
            pltpu.VMEM((8, _D), jnp.float32),
            pltpu.SemaphoreType.DMA((_SNBUF,)),
            pltpu.SemaphoreType.DMA((_SNBUF,)),
        ],
    )(v0, a0, W, b.reshape(1, _E))

    combined = pl.pallas_call(
        _copy_body,
        grid_spec=pltpu.PrefetchScalarGridSpec(
            num_scalar_prefetch=1,
            grid=(1,),
            in_specs=[
                pl.BlockSpec(memory_space=pl.ANY),
                pl.BlockSpec(memory_space=pl.ANY),
                pl.BlockSpec(memory_space=pl.ANY),
            ],
            out_specs=pl.BlockSpec(memory_space=pl.ANY),
            scratch_shapes=[
                pltpu.VMEM((_NBUF, _CH, _D), jnp.float32),
                pltpu.SemaphoreType.DMA((_NBUF,)),
                pltpu.SemaphoreType.DMA((_NBUF,)),
            ],
        ),
        out_shape=jax.ShapeDtypeStruct((_B, _S, _D), jnp.float32),
    )(logits.reshape(_B * _E), v, a, av)

    return combined, logits


# speedup vs baseline: 21.4569x; 1.0039x over previous
import jax
import jax.numpy as jnp
from jax.experimental import pallas as pl
from jax.experimental.pallas import tpu as pltpu

_B, _S, _D, _E = 4, 2048, 1024, 3
_S_BLK = 1024
_S_BLKS = _S // _S_BLK

_CH = 256
_CHUNKS = _S // _CH
_TOTAL = _B * _CHUNKS
_NBUF = 8
_LAG = 4


_SCH = 256
_SCHUNKS = _S // _SCH
_STOTAL = _B * _SCHUNKS
_SNBUF = 6


def _stats_body(v0_ref, a0_ref, w_ref, bias_ref, logits_ref,
                vbuf_ref, abuf_ref, acc_ref, vsem, asem):
    def read_start(t):
        b, j = divmod(t, _SCHUNKS)
        rows = pl.ds(j * _SCH, _SCH)
        slot = t % _SNBUF
        pltpu.make_async_copy(
            v0_ref.at[b, rows, :], vbuf_ref.at[slot], vsem.at[slot]
        ).start()
        pltpu.make_async_copy(
            a0_ref.at[b, rows, :], abuf_ref.at[slot], asem.at[slot]
        ).start()

    def read_wait(t):
        b, j = divmod(t, _SCHUNKS)
        rows = pl.ds(j * _SCH, _SCH)
        slot = t % _SNBUF
        pltpu.make_async_copy(
            v0_ref.at[b, rows, :], vbuf_ref.at[slot], vsem.at[slot]
        ).wait()
        pltpu.make_async_copy(
            a0_ref.at[b, rows, :], abuf_ref.at[slot], asem.at[slot]
        ).wait()

    for t in range(min(_SNBUF, _STOTAL)):
        read_start(t)

    def chunk_sums(buf, slot):
        x0 = buf[slot, 0:8, :]
        s, q = x0, x0 * x0
        for k in range(1, _SCH // 8):
            x = buf[slot, 8 * k:8 * (k + 1), :]
            s = s + x
            q = q + x * x
        return s, q

    for t in range(_STOTAL):
        b, j = divmod(t, _SCHUNKS)
        slot = t % _SNBUF
        read_wait(t)
        sv, qv = chunk_sums(vbuf_ref, slot)
        sa, qa = chunk_sums(abuf_ref, slot)
        if j == 0:
            acc_ref[0:8, :] = sv
            acc_ref[8:16, :] = qv
            acc_ref[16:24, :] = sa
            acc_ref[24:32, :] = qa
        else:
            acc_ref[0:8, :] += sv
            acc_ref[8:16, :] += qv
            acc_ref[16:24, :] += sa
            acc_ref[24:32, :] += qa
        if t + _SNBUF < _STOTAL:
            read_start(t + _SNBUF)
        if j == _SCHUNKS - 1:
            inv_s = 1.0 / _S
            inv_n1 = 1.0 / (_S - 1)
            s_v = jnp.sum(acc_ref[0:8, :], axis=0, keepdims=True)
            q_v = jnp.sum(acc_ref[8:16, :], axis=0, keepdims=True)
            s_a = jnp.sum(acc_ref[16:24, :], axis=0, keepdims=True)
            q_a = jnp.sum(acc_ref[24:32, :], axis=0, keepdims=True)
            mean_v = s_v * inv_s
            var_v = (q_v - _S * mean_v * mean_v) * inv_n1
            mean_a = s_a * inv_s
            var_a = (q_a - _S * mean_a * mean_a) * inv_n1
            feats = jnp.concatenate(
                [mean_v, jnp.sqrt(var_v), mean_a, jnp.sqrt(var_a)], axis=1
            )
            logits = jnp.sum(w_ref[...] * feats, axis=1) + bias_ref[0]
            logits_ref[b, :] = logits


def _argmax3(lg_ref, b):
    l0 = lg_ref[3 * b]
    l1 = lg_ref[3 * b + 1]
    l2 = lg_ref[3 * b + 2]
    i01 = jnp.where(l1 > l0, 1, 0)
    m01 = jnp.maximum(l0, l1)
    return jnp.where(l2 > m01, 2, i01)


def _copy_body(lg_ref, v_ref, a_ref, av_ref, out_ref, buf_ref, rsem, wsem):
    es = [_argmax3(lg_ref, b) for b in range(_B)]

    def chunk(t):
        b, j = divmod(t, _CHUNKS)
        return b, pl.ds(j * _CH, _CH), t % _NBUF

    def read_start(t):
        b, rows, slot = chunk(t)
        for e, src in ((0, v_ref), (1, a_ref), (2, av_ref)):
            @pl.when(es[b] == e)
            def _(src=src):
                pltpu.make_async_copy(
                    src.at[b, rows, :], buf_ref.at[slot], rsem.at[slot]
                ).start()

    def read_wait(t):
        b, rows, slot = chunk(t)
        pltpu.make_async_copy(
            v_ref.at[b, rows, :], buf_ref.at[slot], rsem.at[slot]
        ).wait()

    def write_copy(t):
        b, rows, slot = chunk(t)
        return pltpu.make_async_copy(
            buf_ref.at[slot], out_ref.at[b, rows, :], wsem.at[slot]
        )

    for t in range(_TOTAL + _LAG):
        if t < _TOTAL:
            if t >= _NBUF:
                write_copy(t - _NBUF).wait()
            read_start(t)
        if t >= _LAG:
            read_wait(t - _LAG)
            write_copy(t - _LAG).start()
    for t in range(_TOTAL - _NBUF, _TOTAL):
        write_copy(t).wait()


def kernel(v0, a0, v, a, av, W, b):
    logits = pl.pallas_call(
        _stats_body,
        grid=(1,),
        in_specs=[
            pl.BlockSpec(memory_space=pl.ANY),
            pl.BlockSpec(memory_space=pl.ANY),
            pl.BlockSpec((_E, 4 * _D), lambda i: (0, 0)),
            pl.BlockSpec((1, _E), lambda i: (0, 0)),
        ],
        out_specs=pl.BlockSpec((_B, _E), lambda i: (0, 0)),
        out_shape=jax.ShapeDtypeStruct((_B, _E), jnp.float32),
        scratch_shapes=[
            pltpu.VMEM((_SNBUF, _SCH, _D), jnp.float32),
            pltpu.VMEM((_SNBUF, _SCH, _D), jnp.float32),
            pltpu.VMEM((32, _D), jnp.float32),
            pltpu.SemaphoreType.DMA((_SNBUF,)),
            pltpu.SemaphoreType.DMA((_SNBUF,)),
        ],
    )(v0, a0, W, b.reshape(1, _E))

    combined = pl.pallas_call(
        _copy_body,
        grid_spec=pltpu.PrefetchScalarGridSpec(
            num_scalar_prefetch=1,
            grid=(1,),
            in_specs=[
                pl.BlockSpec(memory_space=pl.ANY),
                pl.BlockSpec(memory_space=pl.ANY),
                pl.BlockSpec(memory_space=pl.ANY),
            ],
            out_specs=pl.BlockSpec(memory_space=pl.ANY),
            scratch_shapes=[
                pltpu.VMEM((_NBUF, _CH, _D), jnp.float32),
                pltpu.SemaphoreType.DMA((_NBUF,)),
                pltpu.SemaphoreType.DMA((_NBUF,)),
            ],
        ),
        out_shape=jax.ShapeDtypeStruct((_B, _S, _D), jnp.float32),
    )(logits.reshape(_B * _E), v, a, av)

    return combined, logits
